# baseline (speedup 1.0000x reference)
import functools

import jax
import jax.numpy as jnp
from jax import lax
from jax.experimental import pallas as pl
from jax.experimental.pallas import tpu as pltpu
from jax.experimental.pallas import tpu_sc as plsc

N = 10000
E = 320000
IN_CH = 128
HID = 256
OUT_CH = 128

NC, NS, L = 2, 16, 16
NW = NC * NS
CHUNK = 128
CH = 79
EPT = CH * CHUNK
EPAD = NW * EPT
NBLK = NW * CH
CH0 = 142
CH1 = 2 * CH - CH0
CHMAX = max(CH0, CH1)
NBLKP = max((NS - 1) * (CH0 + CH1) + CH0 + CHMAX, NBLK)
NPAD = 10240
RPT = NPAD // NS
BR = 1000
HW = 64

_mesh = plsc.VectorSubcoreMesh(
    core_axis_name="c", subcore_axis_name="s", num_cores=NC, num_subcores=NS)


@functools.partial(
    pl.kernel,
    out_type=jax.ShapeDtypeStruct((NC, NPAD, L), jnp.float32),
    mesh=_mesh,
    scratch_types=[
        pltpu.VMEM((CH, CHUNK), jnp.int32),
        pltpu.VMEM((CHUNK, L), jnp.float32),
        pltpu.VMEM((CHUNK, L), jnp.float32),
        pltpu.VMEM_SHARED((NPAD, L), jnp.float32),
    ],
    compiler_params=pltpu.CompilerParams(use_tc_tiling_on_sc=False),
)
def _deg_kernel(dstb, ones_hbm, zeros_hbm, out, dst_v, ones_v, zb, acc):
    c = lax.axis_index("c")
    s = lax.axis_index("s")
    wid = s * NC + c
    pltpu.sync_copy(dstb.at[pl.ds(wid * CH, CH)], dst_v)
    pltpu.sync_copy(ones_hbm, ones_v)
    pltpu.sync_copy(zeros_hbm, zb)
    for k in range(RPT // CHUNK):
        pltpu.sync_copy(zb, acc.at[pl.ds(s * RPT + k * CHUNK, CHUNK)])
    plsc.subcore_barrier()

    def body(j, carry):
        pltpu.sync_copy(ones_v, acc.at[dst_v.at[j]], add=True)
        return carry

    lax.fori_loop(0, CH, body, 0)
    plsc.subcore_barrier()
    pltpu.sync_copy(acc.at[pl.ds(s * RPT, RPT)], out.at[c, pl.ds(s * RPT, RPT)])


@functools.partial(
    pl.kernel,
    out_type=jax.ShapeDtypeStruct((NC, 2, NPAD, HW), jnp.float32),
    mesh=_mesh,
    scratch_types=[
        pltpu.VMEM((CHMAX, CHUNK), jnp.int32),
        pltpu.VMEM((CHMAX, CHUNK), jnp.int32),
        pltpu.VMEM((4, CHUNK, HW), jnp.float32),
        pltpu.VMEM((CHUNK, HW), jnp.float32),
        pltpu.VMEM_SHARED((NPAD, HW), jnp.float32),
        pltpu.SemaphoreType.DMA,
        pltpu.SemaphoreType.DMA,
    ],
    compiler_params=pltpu.CompilerParams(use_tc_tiling_on_sc=False),
)
def _agg_kernel(srcb, dstb, vp_lo, vp_hi, zeros_hbm, out, src_v, dst_v, rows,
                zb, acc, gsem, ssem):
    c = lax.axis_index("c")
    s = lax.axis_index("s")
    base = s * (CH0 + CH1) + c * CH0
    cnt = CH0 + c * (CH1 - CH0)
    pltpu.sync_copy(srcb.at[pl.ds(base, CHMAX)], src_v)
    pltpu.sync_copy(dstb.at[pl.ds(base, CHMAX)], dst_v)
    pltpu.sync_copy(zeros_hbm, zb)
    for h, vp in ((0, vp_lo), (1, vp_hi)):
        for k in range(RPT // CHUNK):
            pltpu.sync_copy(zb, acc.at[pl.ds(s * RPT + k * CHUNK, CHUNK)])
        plsc.subcore_barrier()

        @pl.when(cnt > 0)
        def _stream():
            for b in range(3):
                pltpu.async_copy(vp.at[src_v.at[b]], rows.at[b], gsem)

            def body(j, carry):
                b = lax.rem(j, 4)
                pltpu.make_async_copy(vp.at[src_v.at[j]], rows.at[b],
                                      gsem).wait()
                pltpu.async_copy(rows.at[b], acc.at[dst_v.at[j]], ssem,
                                 add=True)

                @pl.when(j >= 1)
                def _():
                    bp = lax.rem(j + 3, 4)
                    pltpu.make_async_copy(rows.at[bp], acc.at[dst_v.at[j - 1]],
                                          ssem).wait()

                @pl.when(j + 3 < cnt)
                def _():
                    pltpu.async_copy(vp.at[src_v.at[j + 3]],
                                     rows.at[lax.rem(j + 3, 4)], gsem)

                return carry

            lax.fori_loop(0, cnt, body, 0)
            pltpu.make_async_copy(rows.at[lax.rem(cnt - 1, 4)],
                                  acc.at[dst_v.at[cnt - 1]], ssem).wait()

        plsc.subcore_barrier()
        pltpu.sync_copy(acc.at[pl.ds(s * RPT, RPT)],
                        out.at[c, h, pl.ds(s * RPT, RPT)])
        plsc.subcore_barrier()


def _scale_kernel(degp_ref, x_ref, vp_ref, dinvb_ref):
    d = degp_ref[0] + degp_ref[1] + 1.0
    dinv = lax.rsqrt(d)
    db = jnp.broadcast_to(dinv[:, :1], (BR, 128))
    dinvb_ref[...] = db
    vp_ref[...] = db * x_ref[...]


def _halves(aggp_ref):
    return jnp.concatenate(
        [aggp_ref[0, 0] + aggp_ref[1, 0], aggp_ref[0, 1] + aggp_ref[1, 1]],
        axis=1)


def _mid_kernel(aggp_ref, vp1_ref, dinvb_ref, w1_ref, b1_ref, w2_ref, vp2_ref):
    a = (_halves(aggp_ref) + vp1_ref[...]) * dinvb_ref[...]
    h = jnp.dot(a, w1_ref[...], preferred_element_type=jnp.float32)
    h = jnp.maximum(h + b1_ref[...], 0.0)
    g = jnp.dot(h, w2_ref[...], preferred_element_type=jnp.float32)
    vp2_ref[...] = g * dinvb_ref[...]


def _final_kernel(aggp_ref, vp2_ref, dinvb_ref, b2_ref, out_ref):
    out_ref[...] = ((_halves(aggp_ref) + vp2_ref[...])
                    * dinvb_ref[...] + b2_ref[...])


def kernel(x, edge_index, W1, b1, W2, b2):
    ei = edge_index.astype(jnp.int32)
    pad = NBLKP * CHUNK - E
    srcp = jnp.concatenate([ei[0], jnp.zeros((pad,), jnp.int32)])
    dstp = jnp.concatenate([ei[1], jnp.full((pad,), N, jnp.int32)])
    srcb = srcp.reshape(NBLKP, CHUNK)
    dstb = dstp.reshape(NBLKP, CHUNK)
    ones16 = jnp.ones((CHUNK, L), jnp.float32)
    zeros16 = jnp.zeros((CHUNK, L), jnp.float32)
    zeros64 = jnp.zeros((CHUNK, HW), jnp.float32)

    degp = _deg_kernel(dstb, ones16, zeros16)

    grid = N // BR
    vp1, dinvb = pl.pallas_call(
        _scale_kernel,
        grid=(grid,),
        in_specs=[
            pl.BlockSpec((2, BR, L), lambda i: (0, i, 0)),
            pl.BlockSpec((BR, 128), lambda i: (i, 0)),
        ],
        out_specs=[
            pl.BlockSpec((BR, 128), lambda i: (i, 0)),
            pl.BlockSpec((BR, 128), lambda i: (i, 0)),
        ],
        out_shape=[
            jax.ShapeDtypeStruct((N, 128), jnp.float32),
            jax.ShapeDtypeStruct((N, 128), jnp.float32),
        ],
    )(degp, x)

    aggp1 = _agg_kernel(srcb, dstb, vp1[:, :HW], vp1[:, HW:], zeros64)

    vp2 = pl.pallas_call(
        _mid_kernel,
        grid=(grid,),
        in_specs=[
            pl.BlockSpec((2, 2, BR, HW), lambda i: (0, 0, i, 0)),
            pl.BlockSpec((BR, 128), lambda i: (i, 0)),
            pl.BlockSpec((BR, 128), lambda i: (i, 0)),
            pl.BlockSpec((IN_CH, HID), lambda i: (0, 0)),
            pl.BlockSpec((1, HID), lambda i: (0, 0)),
            pl.BlockSpec((HID, OUT_CH), lambda i: (0, 0)),
        ],
        out_specs=pl.BlockSpec((BR, 128), lambda i: (i, 0)),
        out_shape=jax.ShapeDtypeStruct((N, 128), jnp.float32),
    )(aggp1, vp1, dinvb, W1, b1.reshape(1, HID), W2)

    aggp2 = _agg_kernel(srcb, dstb, vp2[:, :HW], vp2[:, HW:], zeros64)

    out = pl.pallas_call(
        _final_kernel,
        grid=(grid,),
        in_specs=[
            pl.BlockSpec((2, 2, BR, HW), lambda i: (0, 0, i, 0)),
            pl.BlockSpec((BR, 128), lambda i: (i, 0)),
            pl.BlockSpec((BR, 128), lambda i: (i, 0)),
            pl.BlockSpec((1, OUT_CH), lambda i: (0, 0)),
        ],
        out_specs=pl.BlockSpec((BR, 128), lambda i: (i, 0)),
        out_shape=jax.ShapeDtypeStruct((N, 128), jnp.float32),
    )(aggp2, vp2, dinvb, b2.reshape(1, OUT_CH))

    return out


# speedup vs baseline: 1.0870x; 1.0870x over previous
import functools

import jax
import jax.numpy as jnp
from jax import lax
from jax.experimental import pallas as pl
from jax.experimental.pallas import tpu as pltpu
from jax.experimental.pallas import tpu_sc as plsc

N = 10000
E = 320000
IN_CH = 128
HID = 256
OUT_CH = 128

NC, NS, L = 2, 16, 16
NW = NC * NS
CHUNK = 128
CH = 79
EPT = CH * CHUNK
EPAD = NW * EPT
NBLK = NW * CH
CH0 = 134
CH1 = 2 * CH - CH0
CHMAX = max(CH0, CH1)
NBLKP = max((NS - 1) * (CH0 + CH1) + CH0 + CHMAX, NBLK)
NPAD = 10240
RPT = NPAD // NS
BR = 1000
HW = 64

_mesh = plsc.VectorSubcoreMesh(
    core_axis_name="c", subcore_axis_name="s", num_cores=NC, num_subcores=NS)


@functools.partial(
    pl.kernel,
    out_type=jax.ShapeDtypeStruct((NC, NPAD, L), jnp.float32),
    mesh=_mesh,
    scratch_types=[
        pltpu.VMEM((CH, CHUNK), jnp.int32),
        pltpu.VMEM((CHUNK, L), jnp.float32),
        pltpu.VMEM((CHUNK, L), jnp.float32),
        pltpu.VMEM_SHARED((NPAD, L), jnp.float32),
    ],
    compiler_params=pltpu.CompilerParams(use_tc_tiling_on_sc=False),
)
def _deg_kernel(dstb, ones_hbm, zeros_hbm, out, dst_v, ones_v, zb, acc):
    c = lax.axis_index("c")
    s = lax.axis_index("s")
    wid = s * NC + c
    pltpu.sync_copy(dstb.at[pl.ds(wid * CH, CH)], dst_v)
    pltpu.sync_copy(ones_hbm, ones_v)
    pltpu.sync_copy(zeros_hbm, zb)
    for k in range(RPT // CHUNK):
        pltpu.sync_copy(zb, acc.at[pl.ds(s * RPT + k * CHUNK, CHUNK)])
    plsc.subcore_barrier()

    def body(j, carry):
        pltpu.sync_copy(ones_v, acc.at[dst_v.at[j]], add=True)
        return carry

    lax.fori_loop(0, CH, body, 0)
    plsc.subcore_barrier()
    pltpu.sync_copy(acc.at[pl.ds(s * RPT, RPT)], out.at[c, pl.ds(s * RPT, RPT)])


@functools.partial(
    pl.kernel,
    out_type=jax.ShapeDtypeStruct((NC, 2, NPAD, HW), jnp.float32),
    mesh=_mesh,
    scratch_types=[
        pltpu.VMEM((CHMAX, CHUNK), jnp.int32),
        pltpu.VMEM((CHMAX, CHUNK), jnp.int32),
        pltpu.VMEM((4, CHUNK, HW), jnp.float32),
        pltpu.VMEM((CHUNK, HW), jnp.float32),
        pltpu.VMEM_SHARED((NPAD, HW), jnp.float32),
        pltpu.SemaphoreType.DMA,
        pltpu.SemaphoreType.DMA,
    ],
    compiler_params=pltpu.CompilerParams(use_tc_tiling_on_sc=False),
)
def _agg_kernel(srcb, dstb, vp_lo, vp_hi, zeros_hbm, out, src_v, dst_v, rows,
                zb, acc, gsem, ssem):
    c = lax.axis_index("c")
    s = lax.axis_index("s")
    base = s * (CH0 + CH1) + c * CH0
    cnt = CH0 + c * (CH1 - CH0)
    pltpu.sync_copy(srcb.at[pl.ds(base, CHMAX)], src_v)
    pltpu.sync_copy(dstb.at[pl.ds(base, CHMAX)], dst_v)
    pltpu.sync_copy(zeros_hbm, zb)
    for h, vp in ((0, vp_lo), (1, vp_hi)):
        for k in range(RPT // CHUNK):
            pltpu.sync_copy(zb, acc.at[pl.ds(s * RPT + k * CHUNK, CHUNK)])
        plsc.subcore_barrier()

        @pl.when(cnt > 0)
        def _stream():
            for b in range(3):
                pltpu.async_copy(vp.at[src_v.at[b]], rows.at[b], gsem)

            def body(j, carry):
                b = lax.rem(j, 4)
                pltpu.make_async_copy(vp.at[src_v.at[j]], rows.at[b],
                                      gsem).wait()
                pltpu.async_copy(rows.at[b], acc.at[dst_v.at[j]], ssem,
                                 add=True)

                @pl.when(j >= 1)
                def _():
                    bp = lax.rem(j + 3, 4)
                    pltpu.make_async_copy(rows.at[bp], acc.at[dst_v.at[j - 1]],
                                          ssem).wait()

                @pl.when(j + 3 < cnt)
                def _():
                    pltpu.async_copy(vp.at[src_v.at[j + 3]],
                                     rows.at[lax.rem(j + 3, 4)], gsem)

                return carry

            lax.fori_loop(0, cnt, body, 0)
            pltpu.make_async_copy(rows.at[lax.rem(cnt - 1, 4)],
                                  acc.at[dst_v.at[cnt - 1]], ssem).wait()

        plsc.subcore_barrier()
        pltpu.sync_copy(acc.at[pl.ds(s * RPT, RPT)],
                        out.at[c, h, pl.ds(s * RPT, RPT)])
        plsc.subcore_barrier()


def _scale_kernel(degp_ref, x_ref, vplo_ref, vphi_ref, dinvb_ref):
    d = degp_ref[0] + degp_ref[1] + 1.0
    dinv = lax.rsqrt(d)
    db = jnp.broadcast_to(dinv[:, :1], (BR, 128))
    dinvb_ref[...] = db
    vp = db * x_ref[...]
    vplo_ref[...] = vp[:, :HW]
    vphi_ref[...] = vp[:, HW:]


def _halves(aggp_ref):
    return jnp.concatenate(
        [aggp_ref[0, 0] + aggp_ref[1, 0], aggp_ref[0, 1] + aggp_ref[1, 1]],
        axis=1)


def _mid_kernel(aggp_ref, vp1lo_ref, vp1hi_ref, dinvb_ref, w1_ref, b1_ref,
                w2_ref, vp2lo_ref, vp2hi_ref):
    vp1 = jnp.concatenate([vp1lo_ref[...], vp1hi_ref[...]], axis=1)
    a = (_halves(aggp_ref) + vp1) * dinvb_ref[...]
    h = jnp.dot(a, w1_ref[...], preferred_element_type=jnp.float32)
    h = jnp.maximum(h + b1_ref[...], 0.0)
    g = jnp.dot(h, w2_ref[...], preferred_element_type=jnp.float32)
    vp2 = g * dinvb_ref[...]
    vp2lo_ref[...] = vp2[:, :HW]
    vp2hi_ref[...] = vp2[:, HW:]


def _final_kernel(aggp_ref, vp2lo_ref, vp2hi_ref, dinvb_ref, b2_ref, out_ref):
    vp2 = jnp.concatenate([vp2lo_ref[...], vp2hi_ref[...]], axis=1)
    out_ref[...] = ((_halves(aggp_ref) + vp2)
                    * dinvb_ref[...] + b2_ref[...])


def kernel(x, edge_index, W1, b1, W2, b2):
    ei = edge_index.astype(jnp.int32)
    pad = NBLKP * CHUNK - E
    srcp = jnp.concatenate([ei[0], jnp.zeros((pad,), jnp.int32)])
    dstp = jnp.concatenate([ei[1], jnp.full((pad,), N, jnp.int32)])
    srcb = srcp.reshape(NBLKP, CHUNK)
    dstb = dstp.reshape(NBLKP, CHUNK)
    ones16 = jnp.ones((CHUNK, L), jnp.float32)
    zeros16 = jnp.zeros((CHUNK, L), jnp.float32)
    zeros64 = jnp.zeros((CHUNK, HW), jnp.float32)

    degp = _deg_kernel(dstb, ones16, zeros16)

    grid = N // BR
    half_spec = pl.BlockSpec((BR, HW), lambda i: (i, 0))
    full_spec = pl.BlockSpec((BR, 128), lambda i: (i, 0))
    half_shape = jax.ShapeDtypeStruct((N, HW), jnp.float32)

    vp1lo, vp1hi, dinvb = pl.pallas_call(
        _scale_kernel,
        grid=(grid,),
        in_specs=[
            pl.BlockSpec((2, BR, L), lambda i: (0, i, 0)),
            full_spec,
        ],
        out_specs=[half_spec, half_spec, full_spec],
        out_shape=[half_shape, half_shape,
                   jax.ShapeDtypeStruct((N, 128), jnp.float32)],
    )(degp, x)

    aggp1 = _agg_kernel(srcb, dstb, vp1lo, vp1hi, zeros64)

    vp2lo, vp2hi = pl.pallas_call(
        _mid_kernel,
        grid=(grid,),
        in_specs=[
            pl.BlockSpec((2, 2, BR, HW), lambda i: (0, 0, i, 0)),
            half_spec,
            half_spec,
            full_spec,
            pl.BlockSpec((IN_CH, HID), lambda i: (0, 0)),
            pl.BlockSpec((1, HID), lambda i: (0, 0)),
            pl.BlockSpec((HID, OUT_CH), lambda i: (0, 0)),
        ],
        out_specs=[half_spec, half_spec],
        out_shape=[half_shape, half_shape],
    )(aggp1, vp1lo, vp1hi, dinvb, W1, b1.reshape(1, HID), W2)

    aggp2 = _agg_kernel(srcb, dstb, vp2lo, vp2hi, zeros64)

    out = pl.pallas_call(
        _final_kernel,
        grid=(grid,),
        in_specs=[
            pl.BlockSpec((2, 2, BR, HW), lambda i: (0, 0, i, 0)),
            half_spec,
            half_spec,
            full_spec,
            pl.BlockSpec((1, OUT_CH), lambda i: (0, 0)),
        ],
        out_specs=full_spec,
        out_shape=jax.ShapeDtypeStruct((N, 128), jnp.float32),
    )(aggp2, vp2lo, vp2hi, dinvb, b2.reshape(1, OUT_CH))

    return out
